# Initial kernel scaffold; baseline (speedup 1.0000x reference)
import jax
import jax.numpy as jnp
from jax.experimental import pallas as pl


def kernel(features, edge_index, edge_values, W_neigh, W_self):
    raise NotImplementedError("write your pallas kernel here")


# speedup vs baseline: 3.0166x; 3.0166x over previous
import functools

import jax
import jax.numpy as jnp
from jax import lax
from jax.experimental import pallas as pl
from jax.experimental.pallas import tpu as pltpu
from jax.experimental.pallas import tpu_sc as plsc

N = 10000
E = 320000
D = 128
DH = D // 2

NC = 2
NS = 16
NW = NC * NS
EW = E // NW
C = 80
NCHUNK = EW // C
NPAD = 10240
RPS = NPAD // NS
ZR = 128


BM = 2000


def _mm_body(x_ref, wn_ref, ws_ref, xw0_ref, xw1_ref, hs_ref):
    x = x_ref[...]
    xw = jnp.dot(x, wn_ref[...], preferred_element_type=jnp.float32)
    xw0_ref[...] = xw[:, :DH]
    xw1_ref[...] = xw[:, DH:]
    hs_ref[...] = jnp.dot(x, ws_ref[...], preferred_element_type=jnp.float32)


def _dense_matmuls(features, W_neigh, W_self):
    return pl.pallas_call(
        _mm_body,
        grid=(N // BM,),
        in_specs=[
            pl.BlockSpec((BM, D), lambda i: (i, 0)),
            pl.BlockSpec((D, D), lambda i: (0, 0)),
            pl.BlockSpec((D, D), lambda i: (0, 0)),
        ],
        out_specs=[
            pl.BlockSpec((BM, DH), lambda i: (i, 0)),
            pl.BlockSpec((BM, DH), lambda i: (i, 0)),
            pl.BlockSpec((BM, D), lambda i: (i, 0)),
        ],
        out_shape=[
            jax.ShapeDtypeStruct((N, DH), jnp.float32),
            jax.ShapeDtypeStruct((N, DH), jnp.float32),
            jax.ShapeDtypeStruct((N, D), jnp.float32),
        ],
    )(features, W_neigh, W_self)


def _sc_body(xw0_hbm, xw1_hbm, src_hbm, dst_hbm, val_hbm, out_hbm,
             src_v, dst_v, val_v, rows, zbuf, acc, gsem):
    cid = lax.axis_index("c")
    sid = lax.axis_index("s")
    wid = cid * NS + sid

    pltpu.sync_copy(src_hbm.at[wid], src_v)
    pltpu.sync_copy(dst_hbm.at[wid], dst_v)
    pltpu.sync_copy(val_hbm.at[wid], val_v)

    def _zero_row(j, carry):
        for k in range(DH // 16):
            zbuf[j, pl.ds(k * 16, 16)] = jnp.zeros((16,), jnp.float32)
        return carry

    lax.fori_loop(0, ZR, _zero_row, 0, unroll=False)
    base = pl.multiple_of(sid * RPS, 8)

    for half, xw_hbm in ((0, xw0_hbm), (1, xw1_hbm)):
        for k in range(RPS // ZR):
            pltpu.sync_copy(zbuf, acc.at[pl.ds(base + k * ZR, ZR)])
        plsc.subcore_barrier()

        def _chunk(i, carry):
            pltpu.async_copy(xw_hbm.at[src_v.at[i]], rows, gsem).wait()

            def _scale(g, c2):
                val16 = val_v[i, pl.ds(g * 16, 16)]
                for j in range(16):
                    vv = jnp.broadcast_to(val16[j], (16,))
                    e = g * 16 + j
                    for k in range(DH // 16):
                        sl = pl.ds(k * 16, 16)
                        rows[e, sl] = rows[e, sl] * vv
                return c2

            lax.fori_loop(0, C // 16, _scale, 0, unroll=False)

            pltpu.sync_copy(rows, acc.at[dst_v.at[i]], add=True)
            return carry

        lax.fori_loop(0, NCHUNK, _chunk, 0, unroll=False)

        plsc.subcore_barrier()

        pltpu.sync_copy(acc.at[pl.ds(base, RPS)],
                        out_hbm.at[cid, half, pl.ds(base, RPS)])


def _sc_scatter(xw0, xw1, src, dst, val):
    mesh = plsc.VectorSubcoreMesh(core_axis_name="c", subcore_axis_name="s")
    return pl.kernel(
        _sc_body,
        out_type=jax.ShapeDtypeStruct((NC, 2, NPAD, DH), jnp.float32),
        mesh=mesh,
        compiler_params=pltpu.CompilerParams(use_tc_tiling_on_sc=False),
        scratch_types=[
            pltpu.VMEM((NCHUNK, C), jnp.int32),
            pltpu.VMEM((NCHUNK, C), jnp.int32),
            pltpu.VMEM((NCHUNK, C), jnp.float32),
            pltpu.VMEM((C, DH), jnp.float32),
            pltpu.VMEM((ZR, DH), jnp.float32),
            pltpu.VMEM_SHARED((NPAD, DH), jnp.float32),
            pltpu.SemaphoreType.DMA,
        ],
    )(xw0, xw1, src, dst, val)


def _combine_body(p_ref, hs_ref, o_ref):
    left = p_ref[0, 0] + p_ref[1, 0]
    right = p_ref[0, 1] + p_ref[1, 1]
    o_ref[...] = jnp.concatenate([left, right], axis=1) + hs_ref[...]


def _combine(partials, h_self):
    return pl.pallas_call(
        _combine_body,
        grid=(N // BM,),
        in_specs=[
            pl.BlockSpec((NC, 2, BM, DH), lambda i: (0, 0, i, 0)),
            pl.BlockSpec((BM, D), lambda i: (i, 0)),
        ],
        out_specs=pl.BlockSpec((BM, D), lambda i: (i, 0)),
        out_shape=jax.ShapeDtypeStruct((N, D), jnp.float32),
    )(partials, h_self)


@jax.jit
def kernel(features, edge_index, edge_values, W_neigh, W_self):
    xw0, xw1, h_self = _dense_matmuls(features, W_neigh, W_self)
    src = edge_index[0].astype(jnp.int32).reshape(NW, NCHUNK, C)
    dst = edge_index[1].astype(jnp.int32).reshape(NW, NCHUNK, C)
    val = edge_values.astype(jnp.float32).reshape(NW, NCHUNK, C)
    partials = _sc_scatter(xw0, xw1, src, dst, val)
    return _combine(partials, h_self)


# speedup vs baseline: 4.6570x; 1.5438x over previous
import functools

import jax
import jax.numpy as jnp
from jax import lax
from jax.experimental import pallas as pl
from jax.experimental.pallas import tpu as pltpu
from jax.experimental.pallas import tpu_sc as plsc

N = 10000
E = 320000
D = 128
DH = D // 2

NC = 2
NS = 16
NW = NC * NS
EW = E // NW
C = 80
NCHUNK = EW // C
NBODY = (NCHUNK - 2) // 3
NPAD = 10240
RPS = NPAD // NS
ZR = 128


BM = 2000


def _mm_body(x_ref, wn_ref, ws_ref, xw0_ref, xw1_ref, hs_ref):
    x = x_ref[...]
    xw = jnp.dot(x, wn_ref[...], preferred_element_type=jnp.float32)
    xw0_ref[...] = xw[:, :DH]
    xw1_ref[...] = xw[:, DH:]
    hs_ref[...] = jnp.dot(x, ws_ref[...], preferred_element_type=jnp.float32)


def _dense_matmuls(features, W_neigh, W_self):
    return pl.pallas_call(
        _mm_body,
        grid=(N // BM,),
        in_specs=[
            pl.BlockSpec((BM, D), lambda i: (i, 0)),
            pl.BlockSpec((D, D), lambda i: (0, 0)),
            pl.BlockSpec((D, D), lambda i: (0, 0)),
        ],
        out_specs=[
            pl.BlockSpec((BM, DH), lambda i: (i, 0)),
            pl.BlockSpec((BM, DH), lambda i: (i, 0)),
            pl.BlockSpec((BM, D), lambda i: (i, 0)),
        ],
        out_shape=[
            jax.ShapeDtypeStruct((N, DH), jnp.float32),
            jax.ShapeDtypeStruct((N, DH), jnp.float32),
            jax.ShapeDtypeStruct((N, D), jnp.float32),
        ],
    )(features, W_neigh, W_self)


def _sc_body(xw0_hbm, xw1_hbm, src_hbm, dst_hbm, val_hbm, out_hbm,
             src_v, dst_v, val_v, b0, b1, b2, zbuf, acc,
             g0, g1, g2, s0, s1, s2):
    cid = lax.axis_index("c")
    sid = lax.axis_index("s")
    wid = cid * NS + sid

    pltpu.sync_copy(src_hbm.at[wid], src_v)
    pltpu.sync_copy(dst_hbm.at[wid], dst_v)
    pltpu.sync_copy(val_hbm.at[wid], val_v)

    def _zero_row(j, carry):
        for k in range(DH // 16):
            zbuf[j, pl.ds(k * 16, 16)] = jnp.zeros((16,), jnp.float32)
        return carry

    lax.fori_loop(0, ZR, _zero_row, 0, unroll=False)
    base = pl.multiple_of(sid * RPS, 8)

    def _scale(buf, i):
        def _scale16(g, c2):
            val16 = val_v[i, pl.ds(g * 16, 16)]
            for j in range(16):
                vv = jnp.broadcast_to(val16[j], (16,))
                e = g * 16 + j
                for k in range(DH // 16):
                    sl = pl.ds(k * 16, 16)
                    buf[e, sl] = buf[e, sl] * vv
            return c2

        lax.fori_loop(0, C // 16, _scale16, 0, unroll=False)

    for half, xw_hbm in ((0, xw0_hbm), (1, xw1_hbm)):
        for k in range(RPS // ZR):
            pltpu.sync_copy(zbuf, acc.at[pl.ds(base + k * ZR, ZR)])
        plsc.subcore_barrier()

        def _gather(i, buf, sem):
            pltpu.async_copy(xw_hbm.at[src_v.at[i]], buf, sem)

        def _scatter(i, buf, sem):
            pltpu.async_copy(buf, acc.at[dst_v.at[i]], sem, add=True)

        def _wait(sem, buf):
            pltpu.make_async_copy(xw_hbm.at[pl.ds(0, C)], buf, sem).wait()

        _gather(0, b0, g0)
        _gather(1, b1, g1)

        def _body(k, carry):
            a = 3 * k

            @pl.when(k > 0)
            def _():
                _wait(s2, b2)

            _gather(a + 2, b2, g2)
            _wait(g0, b0)
            _scale(b0, a)
            _scatter(a, b0, s0)

            _wait(g1, b1)
            _scale(b1, a + 1)
            _scatter(a + 1, b1, s1)

            _wait(s0, b0)
            _gather(a + 3, b0, g0)
            _wait(g2, b2)
            _scale(b2, a + 2)
            _scatter(a + 2, b2, s2)

            _wait(s1, b1)
            _gather(a + 4, b1, g1)
            return carry

        lax.fori_loop(0, NBODY, _body, 0, unroll=False)

        _wait(g0, b0)
        _scale(b0, NCHUNK - 2)
        _scatter(NCHUNK - 2, b0, s0)
        _wait(g1, b1)
        _scale(b1, NCHUNK - 1)
        _scatter(NCHUNK - 1, b1, s1)
        _wait(s2, b2)
        _wait(s0, b0)
        _wait(s1, b1)

        plsc.subcore_barrier()

        pltpu.sync_copy(acc.at[pl.ds(base, RPS)],
                        out_hbm.at[cid, half, pl.ds(base, RPS)])


def _sc_scatter(xw0, xw1, src, dst, val):
    mesh = plsc.VectorSubcoreMesh(core_axis_name="c", subcore_axis_name="s")
    return pl.kernel(
        _sc_body,
        out_type=jax.ShapeDtypeStruct((NC, 2, NPAD, DH), jnp.float32),
        mesh=mesh,
        compiler_params=pltpu.CompilerParams(use_tc_tiling_on_sc=False),
        scratch_types=[
            pltpu.VMEM((NCHUNK, C), jnp.int32),
            pltpu.VMEM((NCHUNK, C), jnp.int32),
            pltpu.VMEM((NCHUNK, C), jnp.float32),
            pltpu.VMEM((C, DH), jnp.float32),
            pltpu.VMEM((C, DH), jnp.float32),
            pltpu.VMEM((C, DH), jnp.float32),
            pltpu.VMEM((ZR, DH), jnp.float32),
            pltpu.VMEM_SHARED((NPAD, DH), jnp.float32),
            pltpu.SemaphoreType.DMA,
            pltpu.SemaphoreType.DMA,
            pltpu.SemaphoreType.DMA,
            pltpu.SemaphoreType.DMA,
            pltpu.SemaphoreType.DMA,
            pltpu.SemaphoreType.DMA,
        ],
    )(xw0, xw1, src, dst, val)


def _combine_body(p_ref, hs_ref, o_ref):
    left = p_ref[0, 0] + p_ref[1, 0]
    right = p_ref[0, 1] + p_ref[1, 1]
    o_ref[...] = jnp.concatenate([left, right], axis=1) + hs_ref[...]


def _combine(partials, h_self):
    return pl.pallas_call(
        _combine_body,
        grid=(N // BM,),
        in_specs=[
            pl.BlockSpec((NC, 2, BM, DH), lambda i: (0, 0, i, 0)),
            pl.BlockSpec((BM, D), lambda i: (i, 0)),
        ],
        out_specs=pl.BlockSpec((BM, D), lambda i: (i, 0)),
        out_shape=jax.ShapeDtypeStruct((N, D), jnp.float32),
    )(partials, h_self)


@jax.jit
def kernel(features, edge_index, edge_values, W_neigh, W_self):
    xw0, xw1, h_self = _dense_matmuls(features, W_neigh, W_self)
    src = edge_index[0].astype(jnp.int32).reshape(NW, NCHUNK, C)
    dst = edge_index[1].astype(jnp.int32).reshape(NW, NCHUNK, C)
    val = edge_values.astype(jnp.float32).reshape(NW, NCHUNK, C)
    partials = _sc_scatter(xw0, xw1, src, dst, val)
    return _combine(partials, h_self)


# speedup vs baseline: 9.0700x; 1.9476x over previous
import functools

import jax
import jax.numpy as jnp
from jax import lax
from jax.experimental import pallas as pl
from jax.experimental.pallas import tpu as pltpu
from jax.experimental.pallas import tpu_sc as plsc

N = 10000
E = 320000
D = 128
DH = D // 2

NC = 2
NS = 16
NW = NC * NS
EW = E // NW
C = 80
NCHUNK = EW // C
NBODY = (NCHUNK - 2) // 3
NPAD = 10240
RPS = NPAD // NS
ZR = 128


BM = 2000


def _mm_body(x_ref, wn_ref, ws_ref, xw0_ref, xw1_ref, hs_ref):
    x = x_ref[...]
    xw = jnp.dot(x, wn_ref[...], preferred_element_type=jnp.float32)
    xw0_ref[...] = xw[:, :DH]
    xw1_ref[...] = xw[:, DH:]
    hs_ref[...] = jnp.dot(x, ws_ref[...], preferred_element_type=jnp.float32)


def _dense_matmuls(features, W_neigh, W_self):
    return pl.pallas_call(
        _mm_body,
        grid=(N // BM,),
        in_specs=[
            pl.BlockSpec((BM, D), lambda i: (i, 0)),
            pl.BlockSpec((D, D), lambda i: (0, 0)),
            pl.BlockSpec((D, D), lambda i: (0, 0)),
        ],
        out_specs=[
            pl.BlockSpec((BM, DH), lambda i: (i, 0)),
            pl.BlockSpec((BM, DH), lambda i: (i, 0)),
            pl.BlockSpec((BM, D), lambda i: (i, 0)),
        ],
        out_shape=[
            jax.ShapeDtypeStruct((N, DH), jnp.float32),
            jax.ShapeDtypeStruct((N, DH), jnp.float32),
            jax.ShapeDtypeStruct((N, D), jnp.float32),
        ],
    )(features, W_neigh, W_self)


def _sc_body(xw0_hbm, xw1_hbm, src_hbm, dst_hbm, val_hbm, out_hbm,
             src_v, dst_v, val_v, b0, b1, b2, zbuf, acc,
             g0, g1, g2, s0, s1, s2):
    cid = lax.axis_index("c")
    sid = lax.axis_index("s")
    wid = cid * NS + sid

    pltpu.sync_copy(src_hbm.at[wid], src_v)
    pltpu.sync_copy(dst_hbm.at[wid], dst_v)
    pltpu.sync_copy(val_hbm.at[wid], val_v)

    def _zero_row(j, carry):
        for k in range(DH // 16):
            zbuf[j, pl.ds(k * 16, 16)] = jnp.zeros((16,), jnp.float32)
        return carry

    lax.fori_loop(0, ZR, _zero_row, 0, unroll=False)
    base = pl.multiple_of(sid * RPS, 8)

    def _scale(buf, i):
        def _scale16(g, c2):
            val16 = val_v[i, pl.ds(g * 16, 16)]
            for j in range(16):
                vv = jnp.broadcast_to(val16[j], (16,))
                e = g * 16 + j
                for k in range(DH // 16):
                    sl = pl.ds(k * 16, 16)
                    buf[e, sl] = buf[e, sl] * vv
            return c2

        lax.fori_loop(0, C // 16, _scale16, 0, unroll=True)

    for half, xw_hbm in ((0, xw0_hbm), (1, xw1_hbm)):
        for k in range(RPS // ZR):
            pltpu.sync_copy(zbuf, acc.at[pl.ds(base + k * ZR, ZR)])
        plsc.subcore_barrier()

        def _gather(i, buf, sem):
            pltpu.async_copy(xw_hbm.at[src_v.at[i]], buf, sem)

        def _scatter(i, buf, sem):
            pltpu.async_copy(buf, acc.at[dst_v.at[i]], sem, add=True)

        def _wait(sem, buf):
            pltpu.make_async_copy(xw_hbm.at[pl.ds(0, C)], buf, sem).wait()

        _gather(0, b0, g0)
        _gather(1, b1, g1)

        def _body(k, carry):
            a = 3 * k

            @pl.when(k > 0)
            def _():
                _wait(s2, b2)

            _gather(a + 2, b2, g2)
            _wait(g0, b0)
            _scale(b0, a)
            _scatter(a, b0, s0)

            _wait(g1, b1)
            _scale(b1, a + 1)
            _scatter(a + 1, b1, s1)

            _wait(s0, b0)
            _gather(a + 3, b0, g0)
            _wait(g2, b2)
            _scale(b2, a + 2)
            _scatter(a + 2, b2, s2)

            _wait(s1, b1)
            _gather(a + 4, b1, g1)
            return carry

        lax.fori_loop(0, NBODY, _body, 0, unroll=False)

        _wait(g0, b0)
        _scale(b0, NCHUNK - 2)
        _scatter(NCHUNK - 2, b0, s0)
        _wait(g1, b1)
        _scale(b1, NCHUNK - 1)
        _scatter(NCHUNK - 1, b1, s1)
        _wait(s2, b2)
        _wait(s0, b0)
        _wait(s1, b1)

        plsc.subcore_barrier()

        pltpu.sync_copy(acc.at[pl.ds(base, RPS)],
                        out_hbm.at[cid, half, pl.ds(base, RPS)])


def _sc_scatter(xw0, xw1, src, dst, val):
    mesh = plsc.VectorSubcoreMesh(core_axis_name="c", subcore_axis_name="s")
    return pl.kernel(
        _sc_body,
        out_type=jax.ShapeDtypeStruct((NC, 2, NPAD, DH), jnp.float32),
        mesh=mesh,
        compiler_params=pltpu.CompilerParams(use_tc_tiling_on_sc=False),
        scratch_types=[
            pltpu.VMEM((NCHUNK, C), jnp.int32),
            pltpu.VMEM((NCHUNK, C), jnp.int32),
            pltpu.VMEM((NCHUNK, C), jnp.float32),
            pltpu.VMEM((C, DH), jnp.float32),
            pltpu.VMEM((C, DH), jnp.float32),
            pltpu.VMEM((C, DH), jnp.float32),
            pltpu.VMEM((ZR, DH), jnp.float32),
            pltpu.VMEM_SHARED((NPAD, DH), jnp.float32),
            pltpu.SemaphoreType.DMA,
            pltpu.SemaphoreType.DMA,
            pltpu.SemaphoreType.DMA,
            pltpu.SemaphoreType.DMA,
            pltpu.SemaphoreType.DMA,
            pltpu.SemaphoreType.DMA,
        ],
    )(xw0, xw1, src, dst, val)


def _combine_body(p_ref, hs_ref, o_ref):
    left = p_ref[0, 0] + p_ref[1, 0]
    right = p_ref[0, 1] + p_ref[1, 1]
    o_ref[...] = jnp.concatenate([left, right], axis=1) + hs_ref[...]


def _combine(partials, h_self):
    return pl.pallas_call(
        _combine_body,
        grid=(N // BM,),
        in_specs=[
            pl.BlockSpec((NC, 2, BM, DH), lambda i: (0, 0, i, 0)),
            pl.BlockSpec((BM, D), lambda i: (i, 0)),
        ],
        out_specs=pl.BlockSpec((BM, D), lambda i: (i, 0)),
        out_shape=jax.ShapeDtypeStruct((N, D), jnp.float32),
    )(partials, h_self)


@jax.jit
def kernel(features, edge_index, edge_values, W_neigh, W_self):
    xw0, xw1, h_self = _dense_matmuls(features, W_neigh, W_self)
    src = edge_index[0].astype(jnp.int32).reshape(NW, NCHUNK, C)
    dst = edge_index[1].astype(jnp.int32).reshape(NW, NCHUNK, C)
    val = edge_values.astype(jnp.float32).reshape(NW, NCHUNK, C)
    partials = _sc_scatter(xw0, xw1, src, dst, val)
    return _combine(partials, h_self)


# speedup vs baseline: 10.8273x; 1.1937x over previous
import functools

import jax
import jax.numpy as jnp
import numpy as np
from jax import lax
from jax.experimental import pallas as pl
from jax.experimental.pallas import tpu as pltpu
from jax.experimental.pallas import tpu_sc as plsc

N = 10000
E = 320000
D = 128
DH = D // 2

NC = 2
NS = 16
NW = NC * NS
EW = E // NW
C = 80
NCHUNK = EW // C
NBODY = (NCHUNK - 5) // 3
NPAD = 10240
RPS = NPAD // NS
ZR = 128

_PERM = np.empty(D, dtype=np.int32)
for _m in range(D // 32):
    for _j in range(16):
        _PERM[32 * _m + 2 * _j] = 32 * _m + _j
        _PERM[32 * _m + 2 * _j + 1] = 32 * _m + 16 + _j


BM = 2000


def _mm_body(x_ref, wn_ref, ws_ref, xw0_ref, xw1_ref, hs_ref):
    x = x_ref[...]
    xw = jnp.dot(x, wn_ref[...], preferred_element_type=jnp.float32)
    xw0_ref[...] = xw[:, :DH].astype(jnp.bfloat16)
    xw1_ref[...] = xw[:, DH:].astype(jnp.bfloat16)
    hs_ref[...] = jnp.dot(x, ws_ref[...], preferred_element_type=jnp.float32)


def _dense_matmuls(features, W_neigh_perm, W_self):
    return pl.pallas_call(
        _mm_body,
        grid=(N // BM,),
        in_specs=[
            pl.BlockSpec((BM, D), lambda i: (i, 0)),
            pl.BlockSpec((D, D), lambda i: (0, 0)),
            pl.BlockSpec((D, D), lambda i: (0, 0)),
        ],
        out_specs=[
            pl.BlockSpec((BM, DH), lambda i: (i, 0)),
            pl.BlockSpec((BM, DH), lambda i: (i, 0)),
            pl.BlockSpec((BM, D), lambda i: (i, 0)),
        ],
        out_shape=[
            jax.ShapeDtypeStruct((N, DH), jnp.bfloat16),
            jax.ShapeDtypeStruct((N, DH), jnp.bfloat16),
            jax.ShapeDtypeStruct((N, D), jnp.float32),
        ],
    )(features, W_neigh_perm, W_self)


def _sc_body(xw0_hbm, xw1_hbm, src_hbm, dst_hbm, val_hbm, out_hbm,
             src_v, dst_v, val_v, gb0, gb1, gb2, sb0, sb1, sb2, zbuf, acc,
             g0, g1, g2, s0, s1, s2):
    cid = lax.axis_index("c")
    sid = lax.axis_index("s")
    wid = cid * NS + sid

    pltpu.sync_copy(src_hbm.at[wid], src_v)
    pltpu.sync_copy(dst_hbm.at[wid], dst_v)
    pltpu.sync_copy(val_hbm.at[wid], val_v)

    def _zero_row(j, carry):
        for k in range(DH // 16):
            zbuf[j, pl.ds(k * 16, 16)] = jnp.zeros((16,), jnp.float32)
        return carry

    lax.fori_loop(0, ZR, _zero_row, 0, unroll=False)
    base = pl.multiple_of(sid * RPS, 8)

    def _scale(gbuf, sbuf, i):
        def _scale16(g, c2):
            val16 = val_v[i, pl.ds(g * 16, 16)]
            for j in range(16):
                vv = jnp.broadcast_to(val16[j], (16,))
                e = g * 16 + j
                for m in range(DH // 32):
                    ab = gbuf[e, pl.ds(m * 32, 32)]
                    a, b = plsc.unpack(ab, format=plsc.PackFormat.INTERLEAVED)
                    sbuf[e, pl.ds(m * 32, 16)] = a * vv
                    sbuf[e, pl.ds(m * 32 + 16, 16)] = b * vv
            return c2

        lax.fori_loop(0, C // 16, _scale16, 0, unroll=True)

    for half, xw_hbm in ((0, xw0_hbm), (1, xw1_hbm)):
        for k in range(RPS // ZR):
            pltpu.sync_copy(zbuf, acc.at[pl.ds(base + k * ZR, ZR)])
        plsc.subcore_barrier()

        def _gather(i, gbuf, sem):
            pltpu.async_copy(xw_hbm.at[src_v.at[i]], gbuf, sem)

        def _scatter(i, sbuf, sem):
            pltpu.async_copy(sbuf, acc.at[dst_v.at[i]], sem, add=True)

        def _wait_g(sem, gbuf):
            pltpu.make_async_copy(xw_hbm.at[pl.ds(0, C)], gbuf, sem).wait()

        def _wait_s(sem, sbuf):
            pltpu.make_async_copy(sbuf, acc.at[pl.ds(0, C)], sem).wait()

        def _step(i, gbuf, sbuf, gsem, ssem, guard_swait, prefetch):
            _wait_g(gsem, gbuf)
            if guard_swait:
                @pl.when(i >= 3)
                def _():
                    _wait_s(ssem, sbuf)
            else:
                _wait_s(ssem, sbuf)
            _scale(gbuf, sbuf, i)
            _scatter(i, sbuf, ssem)
            if prefetch:
                _gather(i + 3, gbuf, gsem)

        _gather(0, gb0, g0)
        _gather(1, gb1, g1)
        _gather(2, gb2, g2)

        def _body(k, carry):
            a = 3 * k
            _step(a, gb0, sb0, g0, s0, True, True)
            _step(a + 1, gb1, sb1, g1, s1, True, True)
            _step(a + 2, gb2, sb2, g2, s2, True, True)
            return carry

        lax.fori_loop(0, NBODY, _body, 0, unroll=False)

        bufs = ((gb0, sb0, g0, s0), (gb1, sb1, g1, s1), (gb2, sb2, g2, s2))
        for i in range(3 * NBODY, NCHUNK):
            gbuf, sbuf, gsem, ssem = bufs[i % 3]
            _step(i, gbuf, sbuf, gsem, ssem, False, i + 3 < NCHUNK)
        for i in range(NCHUNK - 3, NCHUNK):
            gbuf, sbuf, gsem, ssem = bufs[i % 3]
            _wait_s(ssem, sbuf)

        plsc.subcore_barrier()

        pltpu.sync_copy(acc.at[pl.ds(base, RPS)],
                        out_hbm.at[cid, half, pl.ds(base, RPS)])


def _sc_scatter(xw0, xw1, src, dst, val):
    mesh = plsc.VectorSubcoreMesh(core_axis_name="c", subcore_axis_name="s")
    return pl.kernel(
        _sc_body,
        out_type=jax.ShapeDtypeStruct((NC, 2, NPAD, DH), jnp.float32),
        mesh=mesh,
        compiler_params=pltpu.CompilerParams(
            use_tc_tiling_on_sc=False, needs_layout_passes=False),
        scratch_types=[
            pltpu.VMEM((NCHUNK, C), jnp.int32),
            pltpu.VMEM((NCHUNK, C), jnp.int32),
            pltpu.VMEM((NCHUNK, C), jnp.float32),
            pltpu.VMEM((C, DH), jnp.bfloat16),
            pltpu.VMEM((C, DH), jnp.bfloat16),
            pltpu.VMEM((C, DH), jnp.bfloat16),
            pltpu.VMEM((C, DH), jnp.float32),
            pltpu.VMEM((C, DH), jnp.float32),
            pltpu.VMEM((C, DH), jnp.float32),
            pltpu.VMEM((ZR, DH), jnp.float32),
            pltpu.VMEM_SHARED((NPAD, DH), jnp.float32),
            pltpu.SemaphoreType.DMA,
            pltpu.SemaphoreType.DMA,
            pltpu.SemaphoreType.DMA,
            pltpu.SemaphoreType.DMA,
            pltpu.SemaphoreType.DMA,
            pltpu.SemaphoreType.DMA,
        ],
    )(xw0, xw1, src, dst, val)


def _combine_body(p_ref, hs_ref, o_ref):
    left = p_ref[0, 0] + p_ref[1, 0]
    right = p_ref[0, 1] + p_ref[1, 1]
    o_ref[...] = jnp.concatenate([left, right], axis=1) + hs_ref[...]


def _combine(partials, h_self):
    return pl.pallas_call(
        _combine_body,
        grid=(N // BM,),
        in_specs=[
            pl.BlockSpec((NC, 2, BM, DH), lambda i: (0, 0, i, 0)),
            pl.BlockSpec((BM, D), lambda i: (i, 0)),
        ],
        out_specs=pl.BlockSpec((BM, D), lambda i: (i, 0)),
        out_shape=jax.ShapeDtypeStruct((N, D), jnp.float32),
    )(partials, h_self)


@jax.jit
def kernel(features, edge_index, edge_values, W_neigh, W_self):
    W_neigh_perm = W_neigh[:, jnp.asarray(_PERM)]
    xw0, xw1, h_self = _dense_matmuls(features, W_neigh_perm, W_self)
    src = edge_index[0].astype(jnp.int32).reshape(NW, NCHUNK, C)
    dst = edge_index[1].astype(jnp.int32).reshape(NW, NCHUNK, C)
    val = edge_values.astype(jnp.float32).reshape(NW, NCHUNK, C)
    partials = _sc_scatter(xw0, xw1, src, dst, val)
    return _combine(partials, h_self)
